# Initial kernel scaffold; baseline (speedup 1.0000x reference)
import jax
import jax.numpy as jnp
from jax.experimental import pallas as pl


def kernel(x, edge_index, edge_attr, batch, W1a, b1a, W1b, b1b, W2a, b2a, W2b, b2b, root1, bias1, root2, bias2):
    raise NotImplementedError("write your pallas kernel here")


# speedup vs baseline: 4.0586x; 4.0586x over previous
import functools

import jax
import jax.numpy as jnp
from jax import lax
from jax.experimental import pallas as pl
from jax.experimental.pallas import tpu as pltpu
from jax.experimental.pallas import tpu_sc as plsc

N_NODES = 10000
NG = 64

NC = 2
NS = 16
NW = NC * NS
CHUNK = 128
ROWS_PER_SUB = N_NODES // NS

EDGE_BLK = 2048
NODE_BLK = 2000


def _f32(x):
    return x.astype(jnp.float32)


@functools.lru_cache(maxsize=None)
def _make_gather(e_pad):
    cw = e_pad // (NW * CHUNK)
    epw = cw * CHUNK
    mesh = plsc.VectorSubcoreMesh(
        core_axis_name="c", subcore_axis_name="s", num_cores=NC,
        num_subcores=NS)

    @functools.partial(
        pl.kernel, mesh=mesh,
        out_type=jax.ShapeDtypeStruct((e_pad, 16), jnp.float32),
        compiler_params=pltpu.CompilerParams(use_tc_tiling_on_sc=False),
        scratch_types=[
            pltpu.VMEM((cw, CHUNK), jnp.int32),
            pltpu.VMEM((epw, 16), jnp.float32),
            pltpu.SemaphoreType.DMA,
        ],
    )
    def gather(table_hbm, idx_hbm, out_hbm, idx_v, rows_v, sem):
        c = lax.axis_index("c")
        s = lax.axis_index("s")
        wid = s * NC + c
        pltpu.sync_copy(idx_hbm.at[pl.ds(wid * cw, cw)], idx_v)

        def fire(j, carry):
            pltpu.async_copy(table_hbm.at[idx_v.at[j]],
                             rows_v.at[pl.ds(j * CHUNK, CHUNK)], sem)
            return carry

        def drain(j, carry):
            pltpu.make_async_copy(table_hbm.at[idx_v.at[j]],
                                  rows_v.at[pl.ds(j * CHUNK, CHUNK)],
                                  sem).wait()
            return carry

        lax.fori_loop(0, cw, fire, 0)
        lax.fori_loop(0, cw, drain, 0)
        pltpu.sync_copy(rows_v, out_hbm.at[pl.ds(wid * epw, epw)])

    return gather


@functools.lru_cache(maxsize=None)
def _make_scatter_add(e_pad):
    cw = e_pad // (NW * CHUNK)
    assert cw % 2 == 0
    hw = cw // 2
    mesh = plsc.VectorSubcoreMesh(
        core_axis_name="c", subcore_axis_name="s", num_cores=NC,
        num_subcores=NS)

    @functools.partial(
        pl.kernel, mesh=mesh,
        out_type=jax.ShapeDtypeStruct((NC, N_NODES, 32), jnp.float32),
        compiler_params=pltpu.CompilerParams(use_tc_tiling_on_sc=False),
        scratch_types=[
            pltpu.VMEM_SHARED((N_NODES, 32), jnp.float32),
            pltpu.VMEM((cw, CHUNK), jnp.int32),
            pltpu.VMEM((hw * CHUNK, 32), jnp.float32),
            pltpu.SemaphoreType.DMA,
            pltpu.SemaphoreType.DMA,
        ],
    )
    def scatter(msg_hbm, idx_hbm, zeros_hbm, out_hbm, acc, idx_v, rows_v,
                sem, sem2):
        c = lax.axis_index("c")
        s = lax.axis_index("s")
        wid = s * NC + c
        pltpu.sync_copy(zeros_hbm.at[pl.ds(s * ROWS_PER_SUB, ROWS_PER_SUB)],
                        acc.at[pl.ds(s * ROWS_PER_SUB, ROWS_PER_SUB)])
        pltpu.sync_copy(idx_hbm.at[pl.ds(wid * cw, cw)], idx_v)
        plsc.subcore_barrier()

        def half(hf, carry):
            base = (wid * cw + hf * hw) * CHUNK
            pltpu.async_copy(msg_hbm.at[pl.ds(base, hw * CHUNK)], rows_v,
                             sem).wait()

            def fire(j, cc):
                pltpu.async_copy(rows_v.at[pl.ds(j * CHUNK, CHUNK)],
                                 acc.at[idx_v.at[hf * hw + j]], sem2,
                                 add=True)
                return cc

            def drain(j, cc):
                pltpu.make_async_copy(rows_v.at[pl.ds(j * CHUNK, CHUNK)],
                                      acc.at[idx_v.at[hf * hw + j]],
                                      sem2).wait()
                return cc

            lax.fori_loop(0, hw, fire, 0)
            lax.fori_loop(0, hw, drain, 0)
            return carry

        lax.fori_loop(0, 2, half, 0)
        plsc.subcore_barrier()
        pltpu.sync_copy(acc.at[pl.ds(s * ROWS_PER_SUB, ROWS_PER_SUB)],
                        out_hbm.at[c, pl.ds(s * ROWS_PER_SUB, ROWS_PER_SUB)])

    return scatter


def _edge_msg_kernel(xj_ref, aT_ref, waT_ref, baT_ref, bsT_ref, out_ref,
                     *, out_c, with_count, n_real):
    blk = xj_ref.shape[0]
    eye_i = jnp.eye(16, dtype=jnp.float32)
    xjT = lax.dot_general(eye_i, xj_ref[...], (((1,), (1,)), ((), ())),
                          preferred_element_type=jnp.float32)
    aT = aT_ref[...]
    hT = jnp.maximum(waT_ref[...] * aT + baT_ref[...], 0.0)
    yT = jnp.dot(bsT_ref[...], xjT,
                 preferred_element_type=jnp.float32)
    msgT = yT[16 * out_c:, :]
    for k in range(16):
        msgT = msgT + hT[k:k + 1, :] * yT[k * out_c:(k + 1) * out_c, :]
    col = pl.program_id(0) * blk + lax.broadcasted_iota(
        jnp.int32, (1, blk), 1)
    validT = (col < n_real).astype(jnp.float32)
    msgT = msgT * validT
    if with_count:
        outT = jnp.concatenate(
            [msgT, validT, jnp.zeros((31 - out_c, blk), jnp.float32)], axis=0)
    else:
        outT = msgT
    eye_o = jnp.eye(32, dtype=jnp.float32)
    out_ref[...] = lax.dot_general(outT, eye_o, (((0,), (0,)), ((), ())),
                                   preferred_element_type=jnp.float32)


def _edge_msg(xj, aT, waT, baT, bstackT, out_c, with_count, n_real):
    e_pad = xj.shape[0]
    grid = e_pad // EDGE_BLK
    kern = functools.partial(_edge_msg_kernel, out_c=out_c,
                             with_count=with_count, n_real=n_real)
    return pl.pallas_call(
        kern,
        grid=(grid,),
        in_specs=[
            pl.BlockSpec((EDGE_BLK, 16), lambda i: (i, 0)),
            pl.BlockSpec((1, EDGE_BLK), lambda i: (0, i)),
            pl.BlockSpec((16, 1), lambda i: (0, 0)),
            pl.BlockSpec((16, 1), lambda i: (0, 0)),
            pl.BlockSpec(bstackT.shape, lambda i: (0, 0)),
        ],
        out_specs=pl.BlockSpec((EDGE_BLK, 32), lambda i: (i, 0)),
        out_shape=jax.ShapeDtypeStruct((e_pad, 32), jnp.float32),
    )(xj, aT, waT, baT, bstackT)


def _node1_kernel(p0_ref, p1_ref, x_ref, root_ref, bias_ref, h_ref, cnt_ref):
    s = p0_ref[...] + p1_ref[...]
    cnt = s[:, 16:17]
    mean = s[:, :16] / jnp.maximum(cnt, 1.0)
    xr = jnp.dot(x_ref[...], root_ref[...], preferred_element_type=jnp.float32)
    h_ref[...] = jnp.maximum(xr + mean + bias_ref[...], 0.0)
    cnt_ref[...] = cnt


def _node1(p0, p1, x, root1, bias1):
    grid = N_NODES // NODE_BLK
    return pl.pallas_call(
        _node1_kernel,
        grid=(grid,),
        in_specs=[
            pl.BlockSpec((NODE_BLK, 32), lambda i: (i, 0)),
            pl.BlockSpec((NODE_BLK, 32), lambda i: (i, 0)),
            pl.BlockSpec((NODE_BLK, 16), lambda i: (i, 0)),
            pl.BlockSpec((16, 16), lambda i: (0, 0)),
            pl.BlockSpec((1, 16), lambda i: (0, 0)),
        ],
        out_specs=[
            pl.BlockSpec((NODE_BLK, 16), lambda i: (i, 0)),
            pl.BlockSpec((NODE_BLK, 1), lambda i: (i, 0)),
        ],
        out_shape=[
            jax.ShapeDtypeStruct((N_NODES, 16), jnp.float32),
            jax.ShapeDtypeStruct((N_NODES, 1), jnp.float32),
        ],
    )(p0, p1, x, root1, bias1)


def _node2_kernel(q0_ref, q1_ref, h_ref, cnt_ref, batch_ref, root_ref,
                  bias_ref, h2_ref, g_ref, gs_ref, gc_ref):
    i = pl.program_id(0)
    nblk = pl.num_programs(0)
    s = q0_ref[...] + q1_ref[...]
    mean = s / jnp.maximum(cnt_ref[...], 1.0)
    hr = jnp.dot(h_ref[...], root_ref[...], preferred_element_type=jnp.float32)
    h2 = jnp.maximum(hr + mean + bias_ref[...], 0.0)
    h2_ref[...] = h2

    @pl.when(i == 0)
    def _init():
        gs_ref[...] = jnp.zeros_like(gs_ref)
        gc_ref[...] = jnp.zeros_like(gc_ref)

    gid = lax.broadcasted_iota(jnp.int32, (1, NG), 1)
    oh = (batch_ref[...] == gid).astype(jnp.float32)
    dn = (((0,), (0,)), ((), ()))
    gs_ref[...] += lax.dot_general(oh, h2, dn,
                                   preferred_element_type=jnp.float32)
    gc_ref[...] += lax.dot_general(
        oh, jnp.ones_like(h2), dn, preferred_element_type=jnp.float32)

    @pl.when(i == nblk - 1)
    def _fin():
        g_ref[...] = gs_ref[...] / jnp.maximum(gc_ref[...], 1.0)


def _node2(q0, q1, h, cnt, batch2d, root2, bias2):
    grid = N_NODES // NODE_BLK
    return pl.pallas_call(
        _node2_kernel,
        grid=(grid,),
        in_specs=[
            pl.BlockSpec((NODE_BLK, 32), lambda i: (i, 0)),
            pl.BlockSpec((NODE_BLK, 32), lambda i: (i, 0)),
            pl.BlockSpec((NODE_BLK, 16), lambda i: (i, 0)),
            pl.BlockSpec((NODE_BLK, 1), lambda i: (i, 0)),
            pl.BlockSpec((NODE_BLK, 1), lambda i: (i, 0)),
            pl.BlockSpec((16, 32), lambda i: (0, 0)),
            pl.BlockSpec((1, 32), lambda i: (0, 0)),
        ],
        out_specs=[
            pl.BlockSpec((NODE_BLK, 32), lambda i: (i, 0)),
            pl.BlockSpec((NG, 32), lambda i: (0, 0)),
        ],
        out_shape=[
            jax.ShapeDtypeStruct((N_NODES, 32), jnp.float32),
            jax.ShapeDtypeStruct((NG, 32), jnp.float32),
        ],
        scratch_shapes=[
            pltpu.VMEM((NG, 32), jnp.float32),
            pltpu.VMEM((NG, 32), jnp.float32),
        ],
    )(q0, q1, h, cnt, batch2d, root2, bias2)


def kernel(x, edge_index, edge_attr, batch, W1a, b1a, W1b, b1b, W2a, b2a,
           W2b, b2b, root1, bias1, root2, bias2):
    n, in_c = x.shape
    e = edge_index.shape[1]
    hid = root1.shape[1]
    out_c = root2.shape[1]

    e_pad = ((e + NW * CHUNK - 1) // (NW * CHUNK)) * (NW * CHUNK)
    pad = e_pad - e
    src = jnp.pad(edge_index[0].astype(jnp.int32), (0, pad)).reshape(-1, CHUNK)
    dst = jnp.pad(edge_index[1].astype(jnp.int32), (0, pad)).reshape(-1, CHUNK)
    aT = jnp.pad(_f32(edge_attr), ((0, pad), (0, 0))).reshape(1, e_pad)
    zeros32 = jnp.zeros((n, 32), jnp.float32)

    bs1T = jnp.concatenate(
        [W1b.reshape(hid, in_c, hid).transpose(1, 0, 2).reshape(in_c, hid * hid),
         b1b.reshape(in_c, hid)], axis=1).T
    bs2T = jnp.concatenate(
        [W2b.reshape(hid, hid, out_c).transpose(1, 0, 2).reshape(hid, hid * out_c),
         b2b.reshape(hid, out_c)], axis=1).T

    gather = _make_gather(e_pad)
    scatter = _make_scatter_add(e_pad)

    xj = gather(_f32(x), src)
    m1 = _edge_msg(xj, aT, W1a.reshape(16, 1), b1a.reshape(16, 1), bs1T,
                   hid, True, e)
    p = scatter(m1, dst, zeros32)
    h, cnt = _node1(p[0], p[1], _f32(x), root1, bias1.reshape(1, 16))

    hj = gather(h, src)
    m2 = _edge_msg(hj, aT, W2a.reshape(16, 1), b2a.reshape(16, 1), bs2T,
                   out_c, False, e)
    q = scatter(m2, dst, zeros32)
    h2, g = _node2(q[0], q[1], h, cnt, batch.astype(jnp.int32).reshape(-1, 1),
                   root2, bias2.reshape(1, 32))

    return (h2, g, batch)
